# baseline (speedup 1.0000x reference)
import functools

import jax
import jax.numpy as jnp
from jax import lax
from jax.experimental import pallas as pl
from jax.experimental.pallas import tpu as pltpu
from jax.experimental.pallas import tpu_sc as plsc

_B = 1024
_D = 32
_N_DB = 100000
_NQ = 4096
_K = 10
_ALPHA = 1.0
_BETA = 1.0
_LAMB = 1e-4
_GAMMA = 0.1
_TAU = 1.0

_CHUNK = 4000
_NSTEP = _N_DB // _CHUNK
_KPAD = 16


def _sc_gather(table, idx):
  n, d = table.shape
  (bsz,) = idx.shape
  info = plsc.get_sparse_core_info()
  nw = info.num_cores * info.num_subcores
  b_per_w = bsz // nw
  mesh = plsc.VectorSubcoreMesh(core_axis_name="c", subcore_axis_name="s")

  @functools.partial(
      pl.kernel,
      mesh=mesh,
      compiler_params=pltpu.CompilerParams(use_tc_tiling_on_sc=False),
      out_type=jax.ShapeDtypeStruct((bsz, d), table.dtype),
      scratch_types=[
          pltpu.VMEM((b_per_w,), jnp.int32),
          pltpu.VMEM((b_per_w, d), table.dtype),
          pltpu.SemaphoreType.DMA,
      ],
  )
  def gk(table_hbm, idx_hbm, out_hbm, idx_v, rows_v, sem):
    wid = lax.axis_index("s") * info.num_cores + lax.axis_index("c")
    base = wid * b_per_w
    pltpu.sync_copy(idx_hbm.at[pl.ds(base, b_per_w)], idx_v)
    pltpu.async_copy(table_hbm.at[idx_v], rows_v, sem).wait()
    pltpu.sync_copy(rows_v, out_hbm.at[pl.ds(base, b_per_w)])

  return gk(table, idx)


def _knn_body(q_ref, w_ref, b_ref, x_ref, tq_out, idx_out, scal_out,
              tq_s, rund_s, runi_s, dch_s):
  step = pl.program_id(0)

  @pl.when(step == 0)
  def _init():
    q = q_ref[...]
    w = w_ref[...]
    bvec = b_ref[...]
    tq = jnp.dot(q, w, preferred_element_type=jnp.float32) + bvec
    tq_s[...] = tq
    tq_out[...] = tq
    rund_s[...] = jnp.full((_B, _KPAD), jnp.inf, jnp.float32)
    runi_s[...] = jnp.zeros((_B, _KPAD), jnp.int32)
    diff = tq - q
    anchor = jnp.sum(diff * diff) / _B
    reg = (jnp.sum(w * w) + jnp.sum(bvec * bvec)) / 2.0
    lane = lax.broadcasted_iota(jnp.int32, (1, 128), 1)
    scal_out[...] = jnp.where(lane == 0, anchor,
                              jnp.where(lane == 1, reg, 0.0))

  tq = tq_s[...]
  xc = x_ref[...]
  qn = jnp.sum(tq * tq, axis=1, keepdims=True)
  g = lax.dot_general(tq, xc, (((1,), (1,)), ((), ())),
                      preferred_element_type=jnp.float32)
  xn = lax.dot_general(jnp.ones((1, _D), jnp.float32), xc * xc,
                       (((1,), (1,)), ((), ())),
                       precision=lax.Precision.HIGHEST,
                       preferred_element_type=jnp.float32)
  dmat = jnp.maximum(qn + xn - 2.0 * g, 0.0)
  dch_s[...] = dmat
  rm0 = jnp.min(dmat, axis=1, keepdims=True)

  colio = lax.broadcasted_iota(jnp.int32, (_B, _CHUNK), 1)
  slotio = lax.broadcasted_iota(jnp.int32, (_B, _KPAD), 1)
  base = step * _CHUNK

  def one_pass(go):
    del go
    d = dch_s[...]
    rm = jnp.min(d, axis=1, keepdims=True)
    am = jnp.min(jnp.where(d == rm, colio, jnp.int32(2 ** 30)),
                 axis=1, keepdims=True)
    rund = rund_s[...]
    runi = runi_s[...]
    thresh = rund[:, _K - 1:_K]
    improve = rm < thresh
    gi = base + am
    pos = jnp.sum((rund <= rm).astype(jnp.int32), axis=1, keepdims=True)
    shift_d = jnp.concatenate([rund[:, :1], rund[:, :-1]], axis=1)
    shift_i = jnp.concatenate([runi[:, :1], runi[:, :-1]], axis=1)
    ins_d = jnp.where(slotio < pos, rund, jnp.where(slotio == pos, rm, shift_d))
    ins_i = jnp.where(slotio < pos, runi, jnp.where(slotio == pos, gi, shift_i))
    upd = improve & (slotio < _K)
    rund_s[...] = jnp.where(upd, ins_d, rund)
    runi_s[...] = jnp.where(upd, ins_i, runi)
    dch_s[...] = jnp.where((colio == am) & improve, jnp.inf, d)
    return jnp.any(improve)

  go0 = jnp.any(rm0 < rund_s[:, _K - 1:_K])
  lax.while_loop(lambda go: go, one_pass, go0)

  @pl.when(step == _NSTEP - 1)
  def _fin():
    idx_out[...] = runi_s[...]


def _knn_topk(q_batch, w, bvec, x):
  return functools.partial(
      pl.pallas_call,
      grid=(_NSTEP,),
      out_shape=[
          jax.ShapeDtypeStruct((_B, _D), jnp.float32),
          jax.ShapeDtypeStruct((_B, _KPAD), jnp.int32),
          jax.ShapeDtypeStruct((1, 128), jnp.float32),
      ],
      in_specs=[
          pl.BlockSpec((_B, _D), lambda i: (0, 0)),
          pl.BlockSpec((_D, _D), lambda i: (0, 0)),
          pl.BlockSpec((1, _D), lambda i: (0, 0)),
          pl.BlockSpec((_CHUNK, _D), lambda i: (i, 0)),
      ],
      out_specs=[
          pl.BlockSpec((_B, _D), lambda i: (0, 0)),
          pl.BlockSpec((_B, _KPAD), lambda i: (0, 0)),
          pl.BlockSpec((1, 128), lambda i: (0, 0)),
      ],
      scratch_shapes=[
          pltpu.VMEM((_B, _D), jnp.float32),
          pltpu.VMEM((_B, _KPAD), jnp.float32),
          pltpu.VMEM((_B, _KPAD), jnp.int32),
          pltpu.VMEM((_B, _CHUNK), jnp.float32),
      ],
  )(_knn_body)(q_batch, w, bvec, x)


def _mmd_body(tq_ref, xb_ref, out_ref, d_s, dg_s):
  tq = tq_ref[...]
  xb = xb_ref[...]
  cmb = jnp.concatenate([tq, xb], axis=0)
  n_col = jnp.sum(cmb * cmb, axis=1, keepdims=True)
  n_row = lax.dot_general(jnp.ones((1, _D), jnp.float32), cmb * cmb,
                          (((1,), (1,)), ((), ())),
                          precision=lax.Precision.HIGHEST,
                          preferred_element_type=jnp.float32)
  m = 2 * _B
  nblk = 8
  rows = m // nblk

  dmin = jnp.inf
  dmax = -jnp.inf
  for t in range(nblk):
    lo, hi = t * rows, (t + 1) * rows
    g = lax.dot_general(cmb[lo:hi, :], cmb[lo:, :], (((1,), (1,)), ((), ())),
                        preferred_element_type=jnp.float32)
    blk = jnp.maximum(n_col[lo:hi, :] + n_row[:, lo:] - 2.0 * g, 0.0)
    d_s[lo:hi, lo:] = blk
    dmin = jnp.minimum(dmin, jnp.min(blk))
    dmax = jnp.maximum(dmax, jnp.max(blk))
    rio = lax.broadcasted_iota(jnp.int32, (rows, rows), 0)
    cio = lax.broadcasted_iota(jnp.int32, (rows, rows), 1)
    dg_s[lo:hi, :] = jnp.sum(
        jnp.where(rio == cio, blk[:, :rows], 0.0), axis=1, keepdims=True)

  tot = m * m

  def count_pair(t0, t1):
    dg = dg_s[...]
    c0 = jnp.sum((dg <= t0).astype(jnp.float32))
    c1 = jnp.sum((dg <= t1).astype(jnp.float32))
    for t in range(nblk):
      lo, hi = t * rows, (t + 1) * rows
      dgb = d_s[lo:hi, lo:hi]
      rio = lax.broadcasted_iota(jnp.int32, (rows, rows), 0)
      cio = lax.broadcasted_iota(jnp.int32, (rows, rows), 1)
      up = cio > rio
      c0 += 2.0 * jnp.sum(((dgb <= t0) & up).astype(jnp.float32))
      c1 += 2.0 * jnp.sum(((dgb <= t1) & up).astype(jnp.float32))
      if hi < m:
        rect = d_s[lo:hi, hi:]
        c0 += 2.0 * jnp.sum((rect <= t0).astype(jnp.float32))
        c1 += 2.0 * jnp.sum((rect <= t1).astype(jnp.float32))
    return c0, c1

  def body(i, state):
    del i
    lo0, hi0, lo1, hi1 = state
    mid0 = lo0 + (hi0 - lo0) // 2
    mid1 = lo1 + (hi1 - lo1) // 2
    t0 = lax.bitcast_convert_type(mid0, jnp.float32)
    t1 = lax.bitcast_convert_type(mid1, jnp.float32)
    c0, c1 = count_pair(t0, t1)
    p0 = c0 >= jnp.float32(tot // 2)
    p1 = c1 >= jnp.float32(tot // 2 + 1)
    return (jnp.where(p0, lo0, mid0 + 1), jnp.where(p0, mid0, hi0),
            jnp.where(p1, lo1, mid1 + 1), jnp.where(p1, mid1, hi1))

  blo = lax.bitcast_convert_type(dmin, jnp.int32)
  bhi = lax.bitcast_convert_type(dmax, jnp.int32)
  lo0, _, lo1, _ = lax.fori_loop(0, 31, body, (blo, bhi, blo, bhi))
  v0 = lax.bitcast_convert_type(lo0, jnp.float32)
  v1 = lax.bitcast_convert_type(lo1, jnp.float32)
  med = (v0 + v1) * 0.5
  sigma_sq = med * 0.5
  sigma_sq = jnp.where(sigma_sq < 1e-6, jnp.float32(1.0), sigma_sq)
  gam = 1.0 / (sigma_sq + 1e-8)

  dg = dg_s[...]
  sxx = jnp.sum(jnp.exp(-gam * dg[:_B, :]))
  syy = jnp.sum(jnp.exp(-gam * dg[_B:, :]))
  sxy = jnp.sum(jnp.exp(-gam * d_s[:_B, _B:]))
  for t in range(nblk // 2):
    lo, hi = t * rows, (t + 1) * rows
    rio = lax.broadcasted_iota(jnp.int32, (rows, rows), 0)
    cio = lax.broadcasted_iota(jnp.int32, (rows, rows), 1)
    up = cio > rio
    sxx += 2.0 * jnp.sum(
        jnp.where(up, jnp.exp(-gam * d_s[lo:hi, lo:hi]), 0.0))
    if hi < _B:
      sxx += 2.0 * jnp.sum(jnp.exp(-gam * d_s[lo:hi, hi:_B]))
    lo2, hi2 = _B + lo, _B + hi
    syy += 2.0 * jnp.sum(
        jnp.where(up, jnp.exp(-gam * d_s[lo2:hi2, lo2:hi2]), 0.0))
    if hi2 < m:
      syy += 2.0 * jnp.sum(jnp.exp(-gam * d_s[lo2:hi2, hi2:]))
  loss = jnp.maximum((sxx + syy - 2.0 * sxy) / float(_B * _B), 0.0)
  lane = lax.broadcasted_iota(jnp.int32, (1, 128), 1)
  out_ref[...] = jnp.where(lane == 0, loss, 0.0)


def _mmd(tq, xb):
  return pl.pallas_call(
      _mmd_body,
      out_shape=jax.ShapeDtypeStruct((1, 128), jnp.float32),
      scratch_shapes=[pltpu.VMEM((2 * _B, 2 * _B), jnp.float32),
                      pltpu.VMEM((2 * _B, 1), jnp.float32)],
  )(tq, xb)


def _kl_body(tq_ref, xn_ref, pi_ref, pw_ref, qi_ref, out_ref):
  tq = tq_ref[...]
  cols = []
  for j in range(_K):
    xj = xn_ref[:, j * _D:(j + 1) * _D]
    dif = tq - xj
    cols.append(jnp.sum(dif * dif, axis=1, keepdims=True))
  l2 = jnp.concatenate(cols, axis=1)
  s = -l2 / _TAU
  smax = jnp.max(s, axis=1, keepdims=True)
  e = jnp.exp(s - smax)
  post_w = e / jnp.sum(e, axis=1, keepdims=True)

  pre_i = pi_ref[...][:, :_K]
  pre_w = pw_ref[...][:, :_K]
  post_i = qi_ref[...][:, :_K]
  c = jnp.concatenate([pre_i, post_i], axis=1)

  p_cols, q_cols, first_cols = [], [], []
  for j in range(2 * _K):
    cj = c[:, j:j + 1]
    p_cols.append(jnp.sum(jnp.where(cj == pre_i, pre_w, 0.0),
                          axis=1, keepdims=True))
    q_cols.append(jnp.sum(jnp.where(cj == post_i, post_w, 0.0),
                          axis=1, keepdims=True))
    if j == 0:
      first_cols.append(jnp.zeros((_B, 1), dtype=jnp.float32))
    else:
      first_cols.append(jnp.sum((c[:, :j] == cj).astype(jnp.float32),
                                axis=1, keepdims=True))
  p_raw = jnp.concatenate(p_cols, axis=1)
  q_raw = jnp.concatenate(q_cols, axis=1)
  first = jnp.concatenate(first_cols, axis=1) == 0.0

  p = jnp.where(first, jnp.maximum(p_raw, 1e-8), jnp.float32(1e-8))
  q = jnp.where(first, jnp.maximum(q_raw, 1e-8), jnp.float32(1e-8))
  p = p / jnp.sum(p, axis=1, keepdims=True)
  q = q / jnp.sum(q, axis=1, keepdims=True)
  kl = jnp.sum(p * (jnp.log(p) - jnp.log(q)), axis=1)
  loss_knn = jnp.sum(kl) / _B
  lane = lax.broadcasted_iota(jnp.int32, (1, 128), 1)
  out_ref[...] = jnp.where(lane == 0, loss_knn, 0.0)


def _kl(tq, xn_flat, pre_i, pre_w, post_i):
  return pl.pallas_call(
      _kl_body,
      out_shape=jax.ShapeDtypeStruct((1, 128), jnp.float32),
  )(tq, xn_flat, pre_i, pre_w, post_i)


def kernel(q_batch, q_indices, X, W, b, pre_indices, pre_weights):
  idx_mmd = jax.random.randint(jax.random.key(42), (_B,), 0, _N_DB)

  pad_w = jnp.zeros((_NQ, _KPAD - _K), jnp.float32)
  packed = jnp.concatenate([
      jnp.concatenate([pre_indices.astype(jnp.float32), pad_w], axis=1),
      jnp.concatenate([pre_weights, pad_w], axis=1),
  ], axis=1)

  x_batch = _sc_gather(X, idx_mmd.astype(jnp.int32))
  pre_rows = _sc_gather(packed, q_indices.astype(jnp.int32))
  pre_i = pre_rows[:, :_KPAD].astype(jnp.int32)
  pre_w = pre_rows[:, _KPAD:]

  tq, post_idx_pad, scal = _knn_topk(q_batch, W, b.reshape(1, _D), X)
  anchor = scal[0, 0]
  reg = scal[0, 1]

  flat_idx = post_idx_pad[:, :_K].reshape(_B * _K)
  xn_flat = _sc_gather(X, flat_idx).reshape(_B, _K * _D)

  loss_dist = _mmd(tq, x_batch)[0, 0]
  loss_knn = _kl(tq, xn_flat, pre_i, pre_w, post_idx_pad)[0, 0]

  total = (_ALPHA * loss_dist + _BETA * loss_knn
           + _LAMB * reg + _GAMMA * anchor)
  return (total, loss_dist, loss_knn, anchor)


# speedup vs baseline: 1.0847x; 1.0847x over previous
import functools

import jax
import jax.numpy as jnp
from jax import lax
from jax.experimental import pallas as pl
from jax.experimental.pallas import tpu as pltpu
from jax.experimental.pallas import tpu_sc as plsc

_B = 1024
_D = 32
_N_DB = 100000
_NQ = 4096
_K = 10
_ALPHA = 1.0
_BETA = 1.0
_LAMB = 1e-4
_GAMMA = 0.1
_TAU = 1.0

_CHUNK = 2000
_NSTEP = _N_DB // _CHUNK
_KPAD = 16


def _sc_gather(table, idx):
  n, d = table.shape
  (bsz,) = idx.shape
  info = plsc.get_sparse_core_info()
  nw = info.num_cores * info.num_subcores
  b_per_w = bsz // nw
  mesh = plsc.VectorSubcoreMesh(core_axis_name="c", subcore_axis_name="s")

  @functools.partial(
      pl.kernel,
      mesh=mesh,
      compiler_params=pltpu.CompilerParams(use_tc_tiling_on_sc=False),
      out_type=jax.ShapeDtypeStruct((bsz, d), table.dtype),
      scratch_types=[
          pltpu.VMEM((b_per_w,), jnp.int32),
          pltpu.VMEM((b_per_w, d), table.dtype),
          pltpu.SemaphoreType.DMA,
      ],
  )
  def gk(table_hbm, idx_hbm, out_hbm, idx_v, rows_v, sem):
    wid = lax.axis_index("s") * info.num_cores + lax.axis_index("c")
    base = wid * b_per_w
    pltpu.sync_copy(idx_hbm.at[pl.ds(base, b_per_w)], idx_v)
    pltpu.async_copy(table_hbm.at[idx_v], rows_v, sem).wait()
    pltpu.sync_copy(rows_v, out_hbm.at[pl.ds(base, b_per_w)])

  return gk(table, idx)


def _knn_body(q_ref, w_ref, b_ref, x_ref, tq_out, idx_out, scal_out,
              tq_s, rund_s, runi_s, dch_s):
  step = pl.program_id(0)

  @pl.when(step == 0)
  def _init():
    q = q_ref[...]
    w = w_ref[...]
    bvec = b_ref[...]
    tq = jnp.dot(q, w, preferred_element_type=jnp.float32) + bvec
    tq_s[...] = tq
    tq_out[...] = tq
    rund_s[...] = jnp.full((_B, _KPAD), jnp.inf, jnp.float32)
    runi_s[...] = jnp.zeros((_B, _KPAD), jnp.int32)
    diff = tq - q
    anchor = jnp.sum(diff * diff) / _B
    reg = (jnp.sum(w * w) + jnp.sum(bvec * bvec)) / 2.0
    lane = lax.broadcasted_iota(jnp.int32, (1, 128), 1)
    scal_out[...] = jnp.where(lane == 0, anchor,
                              jnp.where(lane == 1, reg, 0.0))

  tq = tq_s[...]
  xc = x_ref[...]
  qn = jnp.sum(tq * tq, axis=1, keepdims=True)
  g = lax.dot_general(tq, xc, (((1,), (1,)), ((), ())),
                      preferred_element_type=jnp.float32)
  xn = lax.dot_general(jnp.ones((1, _D), jnp.float32), xc * xc,
                       (((1,), (1,)), ((), ())),
                       precision=lax.Precision.HIGHEST,
                       preferred_element_type=jnp.float32)
  dmat = jnp.maximum(qn + xn - 2.0 * g, 0.0)
  dch_s[...] = dmat
  rm0 = jnp.min(dmat, axis=1, keepdims=True)

  colio = lax.broadcasted_iota(jnp.int32, (_B, _CHUNK), 1)
  slotio = lax.broadcasted_iota(jnp.int32, (_B, _KPAD), 1)
  base = step * _CHUNK

  def one_pass(go):
    del go
    d = dch_s[...]
    rm = jnp.min(d, axis=1, keepdims=True)
    am = jnp.min(jnp.where(d == rm, colio, jnp.int32(2 ** 30)),
                 axis=1, keepdims=True)
    rund = rund_s[...]
    runi = runi_s[...]
    thresh = rund[:, _K - 1:_K]
    improve = rm < thresh
    gi = base + am
    pos = jnp.sum((rund <= rm).astype(jnp.int32), axis=1, keepdims=True)
    shift_d = jnp.concatenate([rund[:, :1], rund[:, :-1]], axis=1)
    shift_i = jnp.concatenate([runi[:, :1], runi[:, :-1]], axis=1)
    ins_d = jnp.where(slotio < pos, rund, jnp.where(slotio == pos, rm, shift_d))
    ins_i = jnp.where(slotio < pos, runi, jnp.where(slotio == pos, gi, shift_i))
    upd = improve & (slotio < _K)
    rund_s[...] = jnp.where(upd, ins_d, rund)
    runi_s[...] = jnp.where(upd, ins_i, runi)
    dch_s[...] = jnp.where((colio == am) & improve, jnp.inf, d)
    return jnp.any(improve)

  go0 = jnp.any(rm0 < rund_s[:, _K - 1:_K])
  lax.while_loop(lambda go: go, one_pass, go0)

  @pl.when(step == _NSTEP - 1)
  def _fin():
    idx_out[...] = runi_s[...]


def _knn_topk(q_batch, w, bvec, x):
  return functools.partial(
      pl.pallas_call,
      grid=(_NSTEP,),
      out_shape=[
          jax.ShapeDtypeStruct((_B, _D), jnp.float32),
          jax.ShapeDtypeStruct((_B, _KPAD), jnp.int32),
          jax.ShapeDtypeStruct((1, 128), jnp.float32),
      ],
      in_specs=[
          pl.BlockSpec((_B, _D), lambda i: (0, 0)),
          pl.BlockSpec((_D, _D), lambda i: (0, 0)),
          pl.BlockSpec((1, _D), lambda i: (0, 0)),
          pl.BlockSpec((_CHUNK, _D), lambda i: (i, 0)),
      ],
      out_specs=[
          pl.BlockSpec((_B, _D), lambda i: (0, 0)),
          pl.BlockSpec((_B, _KPAD), lambda i: (0, 0)),
          pl.BlockSpec((1, 128), lambda i: (0, 0)),
      ],
      scratch_shapes=[
          pltpu.VMEM((_B, _D), jnp.float32),
          pltpu.VMEM((_B, _KPAD), jnp.float32),
          pltpu.VMEM((_B, _KPAD), jnp.int32),
          pltpu.VMEM((_B, _CHUNK), jnp.float32),
      ],
  )(_knn_body)(q_batch, w, bvec, x)


def _mmd_body(tq_ref, xb_ref, out_ref, d_s, dg_s):
  tq = tq_ref[...]
  xb = xb_ref[...]
  cmb = jnp.concatenate([tq, xb], axis=0)
  n_col = jnp.sum(cmb * cmb, axis=1, keepdims=True)
  n_row = lax.dot_general(jnp.ones((1, _D), jnp.float32), cmb * cmb,
                          (((1,), (1,)), ((), ())),
                          precision=lax.Precision.HIGHEST,
                          preferred_element_type=jnp.float32)
  m = 2 * _B
  nblk = 8
  rows = m // nblk

  dmin = jnp.inf
  dmax = -jnp.inf
  for t in range(nblk):
    lo, hi = t * rows, (t + 1) * rows
    g = lax.dot_general(cmb[lo:hi, :], cmb[lo:, :], (((1,), (1,)), ((), ())),
                        preferred_element_type=jnp.float32)
    blk = jnp.maximum(n_col[lo:hi, :] + n_row[:, lo:] - 2.0 * g, 0.0)
    d_s[lo:hi, lo:] = blk
    dmin = jnp.minimum(dmin, jnp.min(blk))
    dmax = jnp.maximum(dmax, jnp.max(blk))
    rio = lax.broadcasted_iota(jnp.int32, (rows, rows), 0)
    cio = lax.broadcasted_iota(jnp.int32, (rows, rows), 1)
    dg_s[lo:hi, :] = jnp.sum(
        jnp.where(rio == cio, blk[:, :rows], 0.0), axis=1, keepdims=True)

  tot = m * m

  def count_pair(t0, t1):
    dg = dg_s[...]
    c0 = jnp.sum((dg <= t0).astype(jnp.float32))
    c1 = jnp.sum((dg <= t1).astype(jnp.float32))
    for t in range(nblk):
      lo, hi = t * rows, (t + 1) * rows
      dgb = d_s[lo:hi, lo:hi]
      rio = lax.broadcasted_iota(jnp.int32, (rows, rows), 0)
      cio = lax.broadcasted_iota(jnp.int32, (rows, rows), 1)
      up = cio > rio
      c0 += 2.0 * jnp.sum(((dgb <= t0) & up).astype(jnp.float32))
      c1 += 2.0 * jnp.sum(((dgb <= t1) & up).astype(jnp.float32))
      if hi < m:
        rect = d_s[lo:hi, hi:]
        c0 += 2.0 * jnp.sum((rect <= t0).astype(jnp.float32))
        c1 += 2.0 * jnp.sum((rect <= t1).astype(jnp.float32))
    return c0, c1

  def body(i, state):
    del i
    lo0, hi0, lo1, hi1 = state
    mid0 = lo0 + (hi0 - lo0) // 2
    mid1 = lo1 + (hi1 - lo1) // 2
    t0 = lax.bitcast_convert_type(mid0, jnp.float32)
    t1 = lax.bitcast_convert_type(mid1, jnp.float32)
    c0, c1 = count_pair(t0, t1)
    p0 = c0 >= jnp.float32(tot // 2)
    p1 = c1 >= jnp.float32(tot // 2 + 1)
    return (jnp.where(p0, lo0, mid0 + 1), jnp.where(p0, mid0, hi0),
            jnp.where(p1, lo1, mid1 + 1), jnp.where(p1, mid1, hi1))

  blo = lax.bitcast_convert_type(dmin, jnp.int32)
  bhi = lax.bitcast_convert_type(dmax, jnp.int32)
  lo0, _, lo1, _ = lax.fori_loop(0, 31, body, (blo, bhi, blo, bhi))
  v0 = lax.bitcast_convert_type(lo0, jnp.float32)
  v1 = lax.bitcast_convert_type(lo1, jnp.float32)
  med = (v0 + v1) * 0.5
  sigma_sq = med * 0.5
  sigma_sq = jnp.where(sigma_sq < 1e-6, jnp.float32(1.0), sigma_sq)
  gam = 1.0 / (sigma_sq + 1e-8)

  dg = dg_s[...]
  sxx = jnp.sum(jnp.exp(-gam * dg[:_B, :]))
  syy = jnp.sum(jnp.exp(-gam * dg[_B:, :]))
  sxy = jnp.sum(jnp.exp(-gam * d_s[:_B, _B:]))
  for t in range(nblk // 2):
    lo, hi = t * rows, (t + 1) * rows
    rio = lax.broadcasted_iota(jnp.int32, (rows, rows), 0)
    cio = lax.broadcasted_iota(jnp.int32, (rows, rows), 1)
    up = cio > rio
    sxx += 2.0 * jnp.sum(
        jnp.where(up, jnp.exp(-gam * d_s[lo:hi, lo:hi]), 0.0))
    if hi < _B:
      sxx += 2.0 * jnp.sum(jnp.exp(-gam * d_s[lo:hi, hi:_B]))
    lo2, hi2 = _B + lo, _B + hi
    syy += 2.0 * jnp.sum(
        jnp.where(up, jnp.exp(-gam * d_s[lo2:hi2, lo2:hi2]), 0.0))
    if hi2 < m:
      syy += 2.0 * jnp.sum(jnp.exp(-gam * d_s[lo2:hi2, hi2:]))
  loss = jnp.maximum((sxx + syy - 2.0 * sxy) / float(_B * _B), 0.0)
  lane = lax.broadcasted_iota(jnp.int32, (1, 128), 1)
  out_ref[...] = jnp.where(lane == 0, loss, 0.0)


def _mmd(tq, xb):
  return pl.pallas_call(
      _mmd_body,
      out_shape=jax.ShapeDtypeStruct((1, 128), jnp.float32),
      scratch_shapes=[pltpu.VMEM((2 * _B, 2 * _B), jnp.float32),
                      pltpu.VMEM((2 * _B, 1), jnp.float32)],
  )(tq, xb)


def _kl_body(tq_ref, xn_ref, pi_ref, pw_ref, qi_ref, out_ref):
  tq = tq_ref[...]
  cols = []
  for j in range(_K):
    xj = xn_ref[:, j * _D:(j + 1) * _D]
    dif = tq - xj
    cols.append(jnp.sum(dif * dif, axis=1, keepdims=True))
  l2 = jnp.concatenate(cols, axis=1)
  s = -l2 / _TAU
  smax = jnp.max(s, axis=1, keepdims=True)
  e = jnp.exp(s - smax)
  post_w = e / jnp.sum(e, axis=1, keepdims=True)

  pre_i = pi_ref[...][:, :_K]
  pre_w = pw_ref[...][:, :_K]
  post_i = qi_ref[...][:, :_K]
  c = jnp.concatenate([pre_i, post_i], axis=1)

  p_cols, q_cols, first_cols = [], [], []
  for j in range(2 * _K):
    cj = c[:, j:j + 1]
    p_cols.append(jnp.sum(jnp.where(cj == pre_i, pre_w, 0.0),
                          axis=1, keepdims=True))
    q_cols.append(jnp.sum(jnp.where(cj == post_i, post_w, 0.0),
                          axis=1, keepdims=True))
    if j == 0:
      first_cols.append(jnp.zeros((_B, 1), dtype=jnp.float32))
    else:
      first_cols.append(jnp.sum((c[:, :j] == cj).astype(jnp.float32),
                                axis=1, keepdims=True))
  p_raw = jnp.concatenate(p_cols, axis=1)
  q_raw = jnp.concatenate(q_cols, axis=1)
  first = jnp.concatenate(first_cols, axis=1) == 0.0

  p = jnp.where(first, jnp.maximum(p_raw, 1e-8), jnp.float32(1e-8))
  q = jnp.where(first, jnp.maximum(q_raw, 1e-8), jnp.float32(1e-8))
  p = p / jnp.sum(p, axis=1, keepdims=True)
  q = q / jnp.sum(q, axis=1, keepdims=True)
  kl = jnp.sum(p * (jnp.log(p) - jnp.log(q)), axis=1)
  loss_knn = jnp.sum(kl) / _B
  lane = lax.broadcasted_iota(jnp.int32, (1, 128), 1)
  out_ref[...] = jnp.where(lane == 0, loss_knn, 0.0)


def _kl(tq, xn_flat, pre_i, pre_w, post_i):
  return pl.pallas_call(
      _kl_body,
      out_shape=jax.ShapeDtypeStruct((1, 128), jnp.float32),
  )(tq, xn_flat, pre_i, pre_w, post_i)


def kernel(q_batch, q_indices, X, W, b, pre_indices, pre_weights):
  idx_mmd = jax.random.randint(jax.random.key(42), (_B,), 0, _N_DB)

  pad_w = jnp.zeros((_NQ, _KPAD - _K), jnp.float32)
  packed = jnp.concatenate([
      jnp.concatenate([pre_indices.astype(jnp.float32), pad_w], axis=1),
      jnp.concatenate([pre_weights, pad_w], axis=1),
  ], axis=1)

  x_batch = _sc_gather(X, idx_mmd.astype(jnp.int32))
  pre_rows = _sc_gather(packed, q_indices.astype(jnp.int32))
  pre_i = pre_rows[:, :_KPAD].astype(jnp.int32)
  pre_w = pre_rows[:, _KPAD:]

  tq, post_idx_pad, scal = _knn_topk(q_batch, W, b.reshape(1, _D), X)
  anchor = scal[0, 0]
  reg = scal[0, 1]

  flat_idx = post_idx_pad[:, :_K].reshape(_B * _K)
  xn_flat = _sc_gather(X, flat_idx).reshape(_B, _K * _D)

  loss_dist = _mmd(tq, x_batch)[0, 0]
  loss_knn = _kl(tq, xn_flat, pre_i, pre_w, post_idx_pad)[0, 0]

  total = (_ALPHA * loss_dist + _BETA * loss_knn
           + _LAMB * reg + _GAMMA * anchor)
  return (total, loss_dist, loss_knn, anchor)


# speedup vs baseline: 1.2134x; 1.1187x over previous
import functools

import jax
import jax.numpy as jnp
from jax import lax
from jax.experimental import pallas as pl
from jax.experimental.pallas import tpu as pltpu
from jax.experimental.pallas import tpu_sc as plsc

_B = 1024
_D = 32
_N_DB = 100000
_NQ = 4096
_K = 10
_ALPHA = 1.0
_BETA = 1.0
_LAMB = 1e-4
_GAMMA = 0.1
_TAU = 1.0

_CHUNK = 2000
_NSTEP = _N_DB // _CHUNK
_KPAD = 16


def _sc_gather(table, idx):
  n, d = table.shape
  (bsz,) = idx.shape
  info = plsc.get_sparse_core_info()
  nw = info.num_cores * info.num_subcores
  b_per_w = bsz // nw
  mesh = plsc.VectorSubcoreMesh(core_axis_name="c", subcore_axis_name="s")

  @functools.partial(
      pl.kernel,
      mesh=mesh,
      compiler_params=pltpu.CompilerParams(use_tc_tiling_on_sc=False),
      out_type=jax.ShapeDtypeStruct((bsz, d), table.dtype),
      scratch_types=[
          pltpu.VMEM((b_per_w,), jnp.int32),
          pltpu.VMEM((b_per_w, d), table.dtype),
          pltpu.SemaphoreType.DMA,
      ],
  )
  def gk(table_hbm, idx_hbm, out_hbm, idx_v, rows_v, sem):
    wid = lax.axis_index("s") * info.num_cores + lax.axis_index("c")
    base = wid * b_per_w
    pltpu.sync_copy(idx_hbm.at[pl.ds(base, b_per_w)], idx_v)
    pltpu.async_copy(table_hbm.at[idx_v], rows_v, sem).wait()
    pltpu.sync_copy(rows_v, out_hbm.at[pl.ds(base, b_per_w)])

  return gk(table, idx)


def _knn_body(q_ref, w_ref, b_ref, x_ref, tq_out, idx_out, scal_out,
              tq_s, rund_s, runi_s, dch_s, rm_s):
  step = pl.program_id(0)

  @pl.when(step == 0)
  def _init():
    q = q_ref[...]
    w = w_ref[...]
    bvec = b_ref[...]
    tq = jnp.dot(q, w, preferred_element_type=jnp.float32) + bvec
    tq_s[...] = tq
    tq_out[...] = tq
    rund_s[...] = jnp.full((_B, _KPAD), jnp.inf, jnp.float32)
    runi_s[...] = jnp.zeros((_B, _KPAD), jnp.int32)
    diff = tq - q
    anchor = jnp.sum(diff * diff) / _B
    reg = (jnp.sum(w * w) + jnp.sum(bvec * bvec)) / 2.0
    lane = lax.broadcasted_iota(jnp.int32, (1, 128), 1)
    scal_out[...] = jnp.where(lane == 0, anchor,
                              jnp.where(lane == 1, reg, 0.0))

  tq = tq_s[...]
  xc = x_ref[...]
  qn = jnp.sum(tq * tq, axis=1, keepdims=True)
  g = lax.dot_general(tq, xc, (((1,), (1,)), ((), ())),
                      preferred_element_type=jnp.float32)
  xn = lax.dot_general(jnp.ones((1, _D), jnp.float32), xc * xc,
                       (((1,), (1,)), ((), ())),
                       precision=lax.Precision.HIGHEST,
                       preferred_element_type=jnp.float32)
  dmat = jnp.maximum(qn + xn - 2.0 * g, 0.0)
  dch_s[...] = dmat
  rm0 = jnp.min(dmat, axis=1, keepdims=True)

  colio = lax.broadcasted_iota(jnp.int32, (_B, _CHUNK), 1)
  slotio = lax.broadcasted_iota(jnp.int32, (_B, _KPAD), 1)
  base = step * _CHUNK

  def one_pass(go):
    del go
    d = dch_s[...]
    rm = rm_s[...]
    am = jnp.min(jnp.where(d == rm, colio, jnp.int32(2 ** 30)),
                 axis=1, keepdims=True)
    rund = rund_s[...]
    runi = runi_s[...]
    thresh = rund[:, _K - 1:_K]
    improve = rm < thresh
    gi = base + am
    pos = jnp.sum((rund <= rm).astype(jnp.int32), axis=1, keepdims=True)
    shift_d = jnp.concatenate([rund[:, :1], rund[:, :-1]], axis=1)
    shift_i = jnp.concatenate([runi[:, :1], runi[:, :-1]], axis=1)
    ins_d = jnp.where(slotio < pos, rund, jnp.where(slotio == pos, rm, shift_d))
    ins_i = jnp.where(slotio < pos, runi, jnp.where(slotio == pos, gi, shift_i))
    upd = improve & (slotio < _K)
    new_d = jnp.where(upd, ins_d, rund)
    rund_s[...] = new_d
    runi_s[...] = jnp.where(upd, ins_i, runi)
    dn = jnp.where((colio == am) & improve, jnp.inf, d)
    dch_s[...] = dn
    rm_n = jnp.min(dn, axis=1, keepdims=True)
    rm_s[...] = rm_n
    return jnp.any(rm_n < new_d[:, _K - 1:_K])

  rm_s[...] = rm0
  go0 = jnp.any(rm0 < rund_s[:, _K - 1:_K])
  lax.while_loop(lambda go: go, one_pass, go0)

  @pl.when(step == _NSTEP - 1)
  def _fin():
    idx_out[...] = runi_s[...]


def _knn_topk(q_batch, w, bvec, x):
  return functools.partial(
      pl.pallas_call,
      grid=(_NSTEP,),
      out_shape=[
          jax.ShapeDtypeStruct((_B, _D), jnp.float32),
          jax.ShapeDtypeStruct((_B, _KPAD), jnp.int32),
          jax.ShapeDtypeStruct((1, 128), jnp.float32),
      ],
      in_specs=[
          pl.BlockSpec((_B, _D), lambda i: (0, 0)),
          pl.BlockSpec((_D, _D), lambda i: (0, 0)),
          pl.BlockSpec((1, _D), lambda i: (0, 0)),
          pl.BlockSpec((_CHUNK, _D), lambda i: (i, 0)),
      ],
      out_specs=[
          pl.BlockSpec((_B, _D), lambda i: (0, 0)),
          pl.BlockSpec((_B, _KPAD), lambda i: (0, 0)),
          pl.BlockSpec((1, 128), lambda i: (0, 0)),
      ],
      scratch_shapes=[
          pltpu.VMEM((_B, _D), jnp.float32),
          pltpu.VMEM((_B, _KPAD), jnp.float32),
          pltpu.VMEM((_B, _KPAD), jnp.int32),
          pltpu.VMEM((_B, _CHUNK), jnp.float32),
          pltpu.VMEM((_B, 1), jnp.float32),
      ],
  )(_knn_body)(q_batch, w, bvec, x)


def _mmd_body(tq_ref, xb_ref, out_ref, d_s, dg_s):
  tq = tq_ref[...]
  xb = xb_ref[...]
  cmb = jnp.concatenate([tq, xb], axis=0)
  n_col = jnp.sum(cmb * cmb, axis=1, keepdims=True)
  n_row = lax.dot_general(jnp.ones((1, _D), jnp.float32), cmb * cmb,
                          (((1,), (1,)), ((), ())),
                          precision=lax.Precision.HIGHEST,
                          preferred_element_type=jnp.float32)
  m = 2 * _B
  nblk = 8
  rows = m // nblk

  dmin = jnp.inf
  dmax = -jnp.inf
  for t in range(nblk):
    lo, hi = t * rows, (t + 1) * rows
    g = lax.dot_general(cmb[lo:hi, :], cmb[lo:, :], (((1,), (1,)), ((), ())),
                        preferred_element_type=jnp.float32)
    blk = jnp.maximum(n_col[lo:hi, :] + n_row[:, lo:] - 2.0 * g, 0.0)
    d_s[lo:hi, lo:] = blk
    dmin = jnp.minimum(dmin, jnp.min(blk))
    dmax = jnp.maximum(dmax, jnp.max(blk))
    rio = lax.broadcasted_iota(jnp.int32, (rows, rows), 0)
    cio = lax.broadcasted_iota(jnp.int32, (rows, rows), 1)
    dg_s[lo:hi, :] = jnp.sum(
        jnp.where(rio == cio, blk[:, :rows], 0.0), axis=1, keepdims=True)

  tot = m * m

  def count_pair(t0, t1):
    dg = dg_s[...]
    c0 = jnp.sum((dg <= t0).astype(jnp.float32))
    c1 = jnp.sum((dg <= t1).astype(jnp.float32))
    for t in range(nblk):
      lo, hi = t * rows, (t + 1) * rows
      dgb = d_s[lo:hi, lo:hi]
      rio = lax.broadcasted_iota(jnp.int32, (rows, rows), 0)
      cio = lax.broadcasted_iota(jnp.int32, (rows, rows), 1)
      up = cio > rio
      c0 += 2.0 * jnp.sum(((dgb <= t0) & up).astype(jnp.float32))
      c1 += 2.0 * jnp.sum(((dgb <= t1) & up).astype(jnp.float32))
      if hi < m:
        rect = d_s[lo:hi, hi:]
        c0 += 2.0 * jnp.sum((rect <= t0).astype(jnp.float32))
        c1 += 2.0 * jnp.sum((rect <= t1).astype(jnp.float32))
    return c0, c1

  def body(i, state):
    del i
    lo0, hi0, lo1, hi1 = state
    mid0 = lo0 + (hi0 - lo0) // 2
    mid1 = lo1 + (hi1 - lo1) // 2
    t0 = lax.bitcast_convert_type(mid0, jnp.float32)
    t1 = lax.bitcast_convert_type(mid1, jnp.float32)
    c0, c1 = count_pair(t0, t1)
    p0 = c0 >= jnp.float32(tot // 2)
    p1 = c1 >= jnp.float32(tot // 2 + 1)
    return (jnp.where(p0, lo0, mid0 + 1), jnp.where(p0, mid0, hi0),
            jnp.where(p1, lo1, mid1 + 1), jnp.where(p1, mid1, hi1))

  blo = lax.bitcast_convert_type(dmin, jnp.int32)
  bhi = lax.bitcast_convert_type(dmax, jnp.int32)
  lo0, _, lo1, _ = lax.fori_loop(0, 31, body, (blo, bhi, blo, bhi))
  v0 = lax.bitcast_convert_type(lo0, jnp.float32)
  v1 = lax.bitcast_convert_type(lo1, jnp.float32)
  med = (v0 + v1) * 0.5
  sigma_sq = med * 0.5
  sigma_sq = jnp.where(sigma_sq < 1e-6, jnp.float32(1.0), sigma_sq)
  gam = 1.0 / (sigma_sq + 1e-8)

  dg = dg_s[...]
  sxx = jnp.sum(jnp.exp(-gam * dg[:_B, :]))
  syy = jnp.sum(jnp.exp(-gam * dg[_B:, :]))
  sxy = jnp.sum(jnp.exp(-gam * d_s[:_B, _B:]))
  for t in range(nblk // 2):
    lo, hi = t * rows, (t + 1) * rows
    rio = lax.broadcasted_iota(jnp.int32, (rows, rows), 0)
    cio = lax.broadcasted_iota(jnp.int32, (rows, rows), 1)
    up = cio > rio
    sxx += 2.0 * jnp.sum(
        jnp.where(up, jnp.exp(-gam * d_s[lo:hi, lo:hi]), 0.0))
    if hi < _B:
      sxx += 2.0 * jnp.sum(jnp.exp(-gam * d_s[lo:hi, hi:_B]))
    lo2, hi2 = _B + lo, _B + hi
    syy += 2.0 * jnp.sum(
        jnp.where(up, jnp.exp(-gam * d_s[lo2:hi2, lo2:hi2]), 0.0))
    if hi2 < m:
      syy += 2.0 * jnp.sum(jnp.exp(-gam * d_s[lo2:hi2, hi2:]))
  loss = jnp.maximum((sxx + syy - 2.0 * sxy) / float(_B * _B), 0.0)
  lane = lax.broadcasted_iota(jnp.int32, (1, 128), 1)
  out_ref[...] = jnp.where(lane == 0, loss, 0.0)


def _mmd(tq, xb):
  return pl.pallas_call(
      _mmd_body,
      out_shape=jax.ShapeDtypeStruct((1, 128), jnp.float32),
      scratch_shapes=[pltpu.VMEM((2 * _B, 2 * _B), jnp.float32),
                      pltpu.VMEM((2 * _B, 1), jnp.float32)],
  )(tq, xb)


def _kl_body(tq_ref, xn_ref, pi_ref, pw_ref, qi_ref, out_ref):
  tq = tq_ref[...]
  cols = []
  for j in range(_K):
    xj = xn_ref[:, j * _D:(j + 1) * _D]
    dif = tq - xj
    cols.append(jnp.sum(dif * dif, axis=1, keepdims=True))
  l2 = jnp.concatenate(cols, axis=1)
  s = -l2 / _TAU
  smax = jnp.max(s, axis=1, keepdims=True)
  e = jnp.exp(s - smax)
  post_w = e / jnp.sum(e, axis=1, keepdims=True)

  pre_i = pi_ref[...][:, :_K]
  pre_w = pw_ref[...][:, :_K]
  post_i = qi_ref[...][:, :_K]
  c = jnp.concatenate([pre_i, post_i], axis=1)

  p_cols, q_cols, first_cols = [], [], []
  for j in range(2 * _K):
    cj = c[:, j:j + 1]
    p_cols.append(jnp.sum(jnp.where(cj == pre_i, pre_w, 0.0),
                          axis=1, keepdims=True))
    q_cols.append(jnp.sum(jnp.where(cj == post_i, post_w, 0.0),
                          axis=1, keepdims=True))
    if j == 0:
      first_cols.append(jnp.zeros((_B, 1), dtype=jnp.float32))
    else:
      first_cols.append(jnp.sum((c[:, :j] == cj).astype(jnp.float32),
                                axis=1, keepdims=True))
  p_raw = jnp.concatenate(p_cols, axis=1)
  q_raw = jnp.concatenate(q_cols, axis=1)
  first = jnp.concatenate(first_cols, axis=1) == 0.0

  p = jnp.where(first, jnp.maximum(p_raw, 1e-8), jnp.float32(1e-8))
  q = jnp.where(first, jnp.maximum(q_raw, 1e-8), jnp.float32(1e-8))
  p = p / jnp.sum(p, axis=1, keepdims=True)
  q = q / jnp.sum(q, axis=1, keepdims=True)
  kl = jnp.sum(p * (jnp.log(p) - jnp.log(q)), axis=1)
  loss_knn = jnp.sum(kl) / _B
  lane = lax.broadcasted_iota(jnp.int32, (1, 128), 1)
  out_ref[...] = jnp.where(lane == 0, loss_knn, 0.0)


def _kl(tq, xn_flat, pre_i, pre_w, post_i):
  return pl.pallas_call(
      _kl_body,
      out_shape=jax.ShapeDtypeStruct((1, 128), jnp.float32),
  )(tq, xn_flat, pre_i, pre_w, post_i)


def kernel(q_batch, q_indices, X, W, b, pre_indices, pre_weights):
  idx_mmd = jax.random.randint(jax.random.key(42), (_B,), 0, _N_DB)

  pad_w = jnp.zeros((_NQ, _KPAD - _K), jnp.float32)
  packed = jnp.concatenate([
      jnp.concatenate([pre_indices.astype(jnp.float32), pad_w], axis=1),
      jnp.concatenate([pre_weights, pad_w], axis=1),
  ], axis=1)

  x_batch = _sc_gather(X, idx_mmd.astype(jnp.int32))
  pre_rows = _sc_gather(packed, q_indices.astype(jnp.int32))
  pre_i = pre_rows[:, :_KPAD].astype(jnp.int32)
  pre_w = pre_rows[:, _KPAD:]

  tq, post_idx_pad, scal = _knn_topk(q_batch, W, b.reshape(1, _D), X)
  anchor = scal[0, 0]
  reg = scal[0, 1]

  flat_idx = post_idx_pad[:, :_K].reshape(_B * _K)
  xn_flat = _sc_gather(X, flat_idx).reshape(_B, _K * _D)

  loss_dist = _mmd(tq, x_batch)[0, 0]
  loss_knn = _kl(tq, xn_flat, pre_i, pre_w, post_idx_pad)[0, 0]

  total = (_ALPHA * loss_dist + _BETA * loss_knn
           + _LAMB * reg + _GAMMA * anchor)
  return (total, loss_dist, loss_knn, anchor)


# speedup vs baseline: 1.2315x; 1.0149x over previous
import functools

import jax
import jax.numpy as jnp
from jax import lax
from jax.experimental import pallas as pl
from jax.experimental.pallas import tpu as pltpu
from jax.experimental.pallas import tpu_sc as plsc

_B = 1024
_D = 32
_N_DB = 100000
_NQ = 4096
_K = 10
_ALPHA = 1.0
_BETA = 1.0
_LAMB = 1e-4
_GAMMA = 0.1
_TAU = 1.0

_CHUNK = 2000
_NSTEP = _N_DB // _CHUNK
_KPAD = 16


def _sc_gather(table, idx):
  n, d = table.shape
  (bsz,) = idx.shape
  info = plsc.get_sparse_core_info()
  nw = info.num_cores * info.num_subcores
  b_per_w = bsz // nw
  mesh = plsc.VectorSubcoreMesh(core_axis_name="c", subcore_axis_name="s")

  @functools.partial(
      pl.kernel,
      mesh=mesh,
      compiler_params=pltpu.CompilerParams(use_tc_tiling_on_sc=False),
      out_type=jax.ShapeDtypeStruct((bsz, d), table.dtype),
      scratch_types=[
          pltpu.VMEM((b_per_w,), jnp.int32),
          pltpu.VMEM((b_per_w, d), table.dtype),
          pltpu.SemaphoreType.DMA,
      ],
  )
  def gk(table_hbm, idx_hbm, out_hbm, idx_v, rows_v, sem):
    wid = lax.axis_index("s") * info.num_cores + lax.axis_index("c")
    base = wid * b_per_w
    pltpu.sync_copy(idx_hbm.at[pl.ds(base, b_per_w)], idx_v)
    pltpu.async_copy(table_hbm.at[idx_v], rows_v, sem).wait()
    pltpu.sync_copy(rows_v, out_hbm.at[pl.ds(base, b_per_w)])

  return gk(table, idx)


def _knn_body(q_ref, w_ref, b_ref, x_ref, tq_out, idx_out, scal_out,
              tq_s, rund_s, runi_s, dch_s, rm_s):
  step = pl.program_id(0)

  @pl.when(step == 0)
  def _init():
    q = q_ref[...]
    w = w_ref[...]
    bvec = b_ref[...]
    tq = jnp.dot(q, w, preferred_element_type=jnp.float32) + bvec
    tq_s[...] = tq
    tq_out[...] = tq
    rund_s[...] = jnp.full((_B, _KPAD), jnp.inf, jnp.float32)
    runi_s[...] = jnp.zeros((_B, _KPAD), jnp.int32)
    diff = tq - q
    anchor = jnp.sum(diff * diff) / _B
    reg = (jnp.sum(w * w) + jnp.sum(bvec * bvec)) / 2.0
    lane = lax.broadcasted_iota(jnp.int32, (1, 128), 1)
    scal_out[...] = jnp.where(lane == 0, anchor,
                              jnp.where(lane == 1, reg, 0.0))

  tq = tq_s[...]
  xc = x_ref[...]
  qn = jnp.sum(tq * tq, axis=1, keepdims=True)
  g = lax.dot_general(tq, xc, (((1,), (1,)), ((), ())),
                      preferred_element_type=jnp.float32)
  xn = lax.dot_general(jnp.ones((1, _D), jnp.float32), xc * xc,
                       (((1,), (1,)), ((), ())),
                       precision=lax.Precision.HIGHEST,
                       preferred_element_type=jnp.float32)
  dmat = jnp.maximum(qn + xn - 2.0 * g, 0.0)
  dch_s[...] = dmat
  rm0 = jnp.min(dmat, axis=1, keepdims=True)

  colio = lax.broadcasted_iota(jnp.int32, (_B, _CHUNK), 1)
  slotio = lax.broadcasted_iota(jnp.int32, (_B, _KPAD), 1)
  base = step * _CHUNK

  def one_pass(go):
    del go
    d = dch_s[...]
    rm = rm_s[...]
    am = jnp.min(jnp.where(d == rm, colio, jnp.int32(2 ** 30)),
                 axis=1, keepdims=True)
    rund = rund_s[...]
    runi = runi_s[...]
    thresh = rund[:, _K - 1:_K]
    improve = rm < thresh
    gi = base + am
    pos = jnp.sum((rund <= rm).astype(jnp.int32), axis=1, keepdims=True)
    shift_d = jnp.concatenate([rund[:, :1], rund[:, :-1]], axis=1)
    shift_i = jnp.concatenate([runi[:, :1], runi[:, :-1]], axis=1)
    ins_d = jnp.where(slotio < pos, rund, jnp.where(slotio == pos, rm, shift_d))
    ins_i = jnp.where(slotio < pos, runi, jnp.where(slotio == pos, gi, shift_i))
    upd = improve & (slotio < _K)
    new_d = jnp.where(upd, ins_d, rund)
    rund_s[...] = new_d
    runi_s[...] = jnp.where(upd, ins_i, runi)
    dn = jnp.where((colio == am) & improve, jnp.inf, d)
    dch_s[...] = dn
    rm_n = jnp.min(dn, axis=1, keepdims=True)
    rm_s[...] = rm_n
    return jnp.any(rm_n < new_d[:, _K - 1:_K])

  rm_s[...] = rm0
  go0 = jnp.any(rm0 < rund_s[:, _K - 1:_K])
  lax.while_loop(lambda go: go, one_pass, go0)

  @pl.when(step == _NSTEP - 1)
  def _fin():
    idx_out[...] = runi_s[...]


def _knn_topk(q_batch, w, bvec, x):
  return functools.partial(
      pl.pallas_call,
      grid=(_NSTEP,),
      out_shape=[
          jax.ShapeDtypeStruct((_B, _D), jnp.float32),
          jax.ShapeDtypeStruct((_B, _KPAD), jnp.int32),
          jax.ShapeDtypeStruct((1, 128), jnp.float32),
      ],
      in_specs=[
          pl.BlockSpec((_B, _D), lambda i: (0, 0)),
          pl.BlockSpec((_D, _D), lambda i: (0, 0)),
          pl.BlockSpec((1, _D), lambda i: (0, 0)),
          pl.BlockSpec((_CHUNK, _D), lambda i: (i, 0)),
      ],
      out_specs=[
          pl.BlockSpec((_B, _D), lambda i: (0, 0)),
          pl.BlockSpec((_B, _KPAD), lambda i: (0, 0)),
          pl.BlockSpec((1, 128), lambda i: (0, 0)),
      ],
      scratch_shapes=[
          pltpu.VMEM((_B, _D), jnp.float32),
          pltpu.VMEM((_B, _KPAD), jnp.float32),
          pltpu.VMEM((_B, _KPAD), jnp.int32),
          pltpu.VMEM((_B, _CHUNK), jnp.float32),
          pltpu.VMEM((_B, 1), jnp.float32),
      ],
  )(_knn_body)(q_batch, w, bvec, x)


def _mmd_body(tq_ref, xb_ref, out_ref, d_s, dg_s):
  tq = tq_ref[...]
  xb = xb_ref[...]
  cmb = jnp.concatenate([tq, xb], axis=0)
  n_col = jnp.sum(cmb * cmb, axis=1, keepdims=True)
  n_row = lax.dot_general(jnp.ones((1, _D), jnp.float32), cmb * cmb,
                          (((1,), (1,)), ((), ())),
                          precision=lax.Precision.HIGHEST,
                          preferred_element_type=jnp.float32)
  m = 2 * _B
  nblk = 8
  rows = m // nblk

  dmin = jnp.inf
  dmax = -jnp.inf
  for t in range(nblk):
    lo, hi = t * rows, (t + 1) * rows
    g = lax.dot_general(cmb[lo:hi, :], cmb[lo:, :], (((1,), (1,)), ((), ())),
                        preferred_element_type=jnp.float32)
    blk = jnp.maximum(n_col[lo:hi, :] + n_row[:, lo:] - 2.0 * g, 0.0)
    d_s[lo:hi, lo:] = blk
    dmin = jnp.minimum(dmin, jnp.min(blk))
    dmax = jnp.maximum(dmax, jnp.max(blk))
    rio = lax.broadcasted_iota(jnp.int32, (rows, rows), 0)
    cio = lax.broadcasted_iota(jnp.int32, (rows, rows), 1)
    dg_s[lo:hi, :] = jnp.sum(
        jnp.where(rio == cio, blk[:, :rows], 0.0), axis=1, keepdims=True)

  tot = m * m

  def count_pair(t0, t1):
    dg = dg_s[...]
    c0 = jnp.sum((dg <= t0).astype(jnp.float32))
    c1 = jnp.sum((dg <= t1).astype(jnp.float32))
    for t in range(nblk):
      lo, hi = t * rows, (t + 1) * rows
      dgb = d_s[lo:hi, lo:hi]
      rio = lax.broadcasted_iota(jnp.int32, (rows, rows), 0)
      cio = lax.broadcasted_iota(jnp.int32, (rows, rows), 1)
      up = cio > rio
      c0 += 2.0 * jnp.sum(((dgb <= t0) & up).astype(jnp.float32))
      c1 += 2.0 * jnp.sum(((dgb <= t1) & up).astype(jnp.float32))
      if hi < m:
        rect = d_s[lo:hi, hi:]
        c0 += 2.0 * jnp.sum((rect <= t0).astype(jnp.float32))
        c1 += 2.0 * jnp.sum((rect <= t1).astype(jnp.float32))
    return c0, c1

  kplus = jnp.float32(tot // 2)

  def body(i, state):
    del i
    lo, hi = state
    r = hi - lo
    m1 = lo + r // 3
    m2 = lo + 2 * (r // 3)
    c1, c2 = count_pair(lax.bitcast_convert_type(m1, jnp.float32),
                        lax.bitcast_convert_type(m2, jnp.float32))
    p1 = c1 >= kplus
    p2 = c2 >= kplus
    newlo = jnp.where(p1, lo, jnp.where(p2, m1 + 1, m2 + 1))
    newhi = jnp.where(p1, m1, jnp.where(p2, m2, hi))
    return newlo, newhi

  blo = lax.bitcast_convert_type(dmin, jnp.int32)
  bhi = lax.bitcast_convert_type(dmax, jnp.int32)
  lo0, _ = lax.fori_loop(0, 23, body, (blo, bhi))
  v0 = lax.bitcast_convert_type(lo0, jnp.float32)

  cnt0 = jnp.sum((dg_s[...] <= v0).astype(jnp.float32))
  vn = jnp.min(jnp.where(dg_s[...] > v0, dg_s[...], jnp.inf))
  for t in range(nblk):
    lo, hi = t * rows, (t + 1) * rows
    dgb = d_s[lo:hi, lo:hi]
    rio = lax.broadcasted_iota(jnp.int32, (rows, rows), 0)
    cio = lax.broadcasted_iota(jnp.int32, (rows, rows), 1)
    up = cio > rio
    cnt0 += 2.0 * jnp.sum(((dgb <= v0) & up).astype(jnp.float32))
    vn = jnp.minimum(vn, jnp.min(jnp.where(up & (dgb > v0), dgb, jnp.inf)))
    if hi < m:
      rect = d_s[lo:hi, hi:]
      cnt0 += 2.0 * jnp.sum((rect <= v0).astype(jnp.float32))
      vn = jnp.minimum(vn, jnp.min(jnp.where(rect > v0, rect, jnp.inf)))
  v1 = jnp.where(cnt0 >= jnp.float32(tot // 2 + 1), v0, vn)
  med = (v0 + v1) * 0.5
  sigma_sq = med * 0.5
  sigma_sq = jnp.where(sigma_sq < 1e-6, jnp.float32(1.0), sigma_sq)
  gam = 1.0 / (sigma_sq + 1e-8)

  dg = dg_s[...]
  sxx = jnp.sum(jnp.exp(-gam * dg[:_B, :]))
  syy = jnp.sum(jnp.exp(-gam * dg[_B:, :]))
  sxy = jnp.sum(jnp.exp(-gam * d_s[:_B, _B:]))
  for t in range(nblk // 2):
    lo, hi = t * rows, (t + 1) * rows
    rio = lax.broadcasted_iota(jnp.int32, (rows, rows), 0)
    cio = lax.broadcasted_iota(jnp.int32, (rows, rows), 1)
    up = cio > rio
    sxx += 2.0 * jnp.sum(
        jnp.where(up, jnp.exp(-gam * d_s[lo:hi, lo:hi]), 0.0))
    if hi < _B:
      sxx += 2.0 * jnp.sum(jnp.exp(-gam * d_s[lo:hi, hi:_B]))
    lo2, hi2 = _B + lo, _B + hi
    syy += 2.0 * jnp.sum(
        jnp.where(up, jnp.exp(-gam * d_s[lo2:hi2, lo2:hi2]), 0.0))
    if hi2 < m:
      syy += 2.0 * jnp.sum(jnp.exp(-gam * d_s[lo2:hi2, hi2:]))
  loss = jnp.maximum((sxx + syy - 2.0 * sxy) / float(_B * _B), 0.0)
  lane = lax.broadcasted_iota(jnp.int32, (1, 128), 1)
  out_ref[...] = jnp.where(lane == 0, loss, 0.0)


def _mmd(tq, xb):
  return pl.pallas_call(
      _mmd_body,
      out_shape=jax.ShapeDtypeStruct((1, 128), jnp.float32),
      scratch_shapes=[pltpu.VMEM((2 * _B, 2 * _B), jnp.float32),
                      pltpu.VMEM((2 * _B, 1), jnp.float32)],
  )(tq, xb)


def _kl_body(tq_ref, xn_ref, pi_ref, pw_ref, qi_ref, out_ref):
  tq = tq_ref[...]
  cols = []
  for j in range(_K):
    xj = xn_ref[:, j * _D:(j + 1) * _D]
    dif = tq - xj
    cols.append(jnp.sum(dif * dif, axis=1, keepdims=True))
  l2 = jnp.concatenate(cols, axis=1)
  s = -l2 / _TAU
  smax = jnp.max(s, axis=1, keepdims=True)
  e = jnp.exp(s - smax)
  post_w = e / jnp.sum(e, axis=1, keepdims=True)

  pre_i = pi_ref[...][:, :_K]
  pre_w = pw_ref[...][:, :_K]
  post_i = qi_ref[...][:, :_K]
  c = jnp.concatenate([pre_i, post_i], axis=1)

  p_cols, q_cols, first_cols = [], [], []
  for j in range(2 * _K):
    cj = c[:, j:j + 1]
    p_cols.append(jnp.sum(jnp.where(cj == pre_i, pre_w, 0.0),
                          axis=1, keepdims=True))
    q_cols.append(jnp.sum(jnp.where(cj == post_i, post_w, 0.0),
                          axis=1, keepdims=True))
    if j == 0:
      first_cols.append(jnp.zeros((_B, 1), dtype=jnp.float32))
    else:
      first_cols.append(jnp.sum((c[:, :j] == cj).astype(jnp.float32),
                                axis=1, keepdims=True))
  p_raw = jnp.concatenate(p_cols, axis=1)
  q_raw = jnp.concatenate(q_cols, axis=1)
  first = jnp.concatenate(first_cols, axis=1) == 0.0

  p = jnp.where(first, jnp.maximum(p_raw, 1e-8), jnp.float32(1e-8))
  q = jnp.where(first, jnp.maximum(q_raw, 1e-8), jnp.float32(1e-8))
  p = p / jnp.sum(p, axis=1, keepdims=True)
  q = q / jnp.sum(q, axis=1, keepdims=True)
  kl = jnp.sum(p * (jnp.log(p) - jnp.log(q)), axis=1)
  loss_knn = jnp.sum(kl) / _B
  lane = lax.broadcasted_iota(jnp.int32, (1, 128), 1)
  out_ref[...] = jnp.where(lane == 0, loss_knn, 0.0)


def _kl(tq, xn_flat, pre_i, pre_w, post_i):
  return pl.pallas_call(
      _kl_body,
      out_shape=jax.ShapeDtypeStruct((1, 128), jnp.float32),
  )(tq, xn_flat, pre_i, pre_w, post_i)


def kernel(q_batch, q_indices, X, W, b, pre_indices, pre_weights):
  idx_mmd = jax.random.randint(jax.random.key(42), (_B,), 0, _N_DB)

  pad_w = jnp.zeros((_NQ, _KPAD - _K), jnp.float32)
  packed = jnp.concatenate([
      jnp.concatenate([pre_indices.astype(jnp.float32), pad_w], axis=1),
      jnp.concatenate([pre_weights, pad_w], axis=1),
  ], axis=1)

  x_batch = _sc_gather(X, idx_mmd.astype(jnp.int32))
  pre_rows = _sc_gather(packed, q_indices.astype(jnp.int32))
  pre_i = pre_rows[:, :_KPAD].astype(jnp.int32)
  pre_w = pre_rows[:, _KPAD:]

  tq, post_idx_pad, scal = _knn_topk(q_batch, W, b.reshape(1, _D), X)
  anchor = scal[0, 0]
  reg = scal[0, 1]

  flat_idx = post_idx_pad[:, :_K].reshape(_B * _K)
  xn_flat = _sc_gather(X, flat_idx).reshape(_B, _K * _D)

  loss_dist = _mmd(tq, x_batch)[0, 0]
  loss_knn = _kl(tq, xn_flat, pre_i, pre_w, post_idx_pad)[0, 0]

  total = (_ALPHA * loss_dist + _BETA * loss_knn
           + _LAMB * reg + _GAMMA * anchor)
  return (total, loss_dist, loss_knn, anchor)
